# baseline (speedup 1.0000x reference)
import jax
import jax.numpy as jnp
from jax.experimental import pallas as pl

N_TOKENS = 32768
D = 768
E = 8
K = 2
BT = 4096
NSLICE = 2
ROWS = BT * NSLICE


def _router_slice(x, w, s, idx_ref, topk_ref, probs_ref):
    logits_t = jax.lax.dot_general(
        w, x, (((1,), (1,)), ((), ())), preferred_element_type=jnp.float32)

    m = jnp.max(logits_t, axis=0, keepdims=True)
    ex = jnp.exp(logits_t - m)
    denom = jnp.sum(ex, axis=0, keepdims=True)
    probs_t = ex / denom

    row = jax.lax.broadcasted_iota(jnp.int32, (E, BT), 0)
    big = jnp.int32(E)
    p1 = jnp.max(probs_t, axis=0, keepdims=True)
    i1 = jnp.min(jnp.where(probs_t == p1, row, big), axis=0, keepdims=True)
    rest = jnp.where(row != i1, probs_t, -1.0)
    p2 = jnp.max(rest, axis=0, keepdims=True)
    i2 = jnp.min(jnp.where(rest == p2, row, big), axis=0, keepdims=True)

    rn = 1.0 / (p1 + p2 + 1e-9)

    sl = pl.ds(s * BT, BT)
    probs_ref[:, sl] = probs_t
    idx_ref[:, sl] = jnp.concatenate([i1, i2], axis=0)
    topk_ref[:, sl] = jnp.concatenate([p1 * rn, p2 * rn], axis=0)


def _body(*refs):
    xs = refs[:NSLICE]
    w_ref = refs[NSLICE]
    idx_ref, topk_ref, probs_ref = refs[NSLICE + 1:]
    w = w_ref[...]
    for s in range(NSLICE):
        _router_slice(xs[s][...], w, s, idx_ref, topk_ref, probs_ref)


@jax.jit
def kernel(x, W_gate, W_noisy):
    grid = (N_TOKENS // ROWS,)
    out_shapes = (
        jax.ShapeDtypeStruct((K, N_TOKENS), jnp.int32),
        jax.ShapeDtypeStruct((K, N_TOKENS), jnp.float32),
        jax.ShapeDtypeStruct((E, N_TOKENS), jnp.float32),
    )
    in_specs = [
        pl.BlockSpec((BT, D), (lambda i, s=s: (i * NSLICE + s, 0)))
        for s in range(NSLICE)
    ] + [pl.BlockSpec((E, D), lambda i: (0, 0))]
    topk_idx, topk_probs, probs = pl.pallas_call(
        _body,
        grid=grid,
        in_specs=in_specs,
        out_specs=(
            pl.BlockSpec((K, ROWS), lambda i: (0, i)),
            pl.BlockSpec((K, ROWS), lambda i: (0, i)),
            pl.BlockSpec((E, ROWS), lambda i: (0, i)),
        ),
        out_shape=out_shapes,
    )(*([x] * NSLICE), W_gate)
    return topk_idx.T, topk_probs.T, probs.T


# speedup vs baseline: 1.1003x; 1.1003x over previous
import jax
import jax.numpy as jnp
from jax import lax
from jax.experimental import pallas as pl
from jax.experimental.pallas import tpu as pltpu

N_TOKENS = 32768
D = 768
E = 8
K = 2
CHUNK = 1024
NBUF = 6
NSTEP = N_TOKENS // CHUNK


def _body(x_hbm, w_ref, idx_ref, topk_ref, probs_ref, xbuf, sems):
    def copy(c, slot):
        pltpu.make_async_copy(
            x_hbm.at[pl.ds(c * CHUNK, CHUNK), :],
            xbuf.at[slot],
            sems.at[slot],
        ).start()

    for c in range(NBUF):
        copy(c, c)

    w = w_ref[...]
    row = jax.lax.broadcasted_iota(jnp.int32, (E, CHUNK), 0)
    big = jnp.int32(E)

    def step(c, carry):
        slot = lax.rem(c, NBUF)
        pltpu.make_async_copy(
            x_hbm.at[pl.ds(0, CHUNK), :], xbuf.at[slot], sems.at[slot]
        ).wait()
        x = xbuf[slot]

        logits_t = jax.lax.dot_general(
            w, x, (((1,), (1,)), ((), ())),
            preferred_element_type=jnp.float32)

        m = jnp.max(logits_t, axis=0, keepdims=True)
        ex = jnp.exp(logits_t - m)
        denom = jnp.sum(ex, axis=0, keepdims=True)
        probs_t = ex / denom

        p1 = jnp.max(probs_t, axis=0, keepdims=True)
        i1 = jnp.min(jnp.where(probs_t == p1, row, big), axis=0,
                     keepdims=True)
        rest = jnp.where(row != i1, probs_t, -1.0)
        p2 = jnp.max(rest, axis=0, keepdims=True)
        i2 = jnp.min(jnp.where(rest == p2, row, big), axis=0, keepdims=True)

        rn = 1.0 / (p1 + p2 + 1e-9)

        sl = pl.ds(c * CHUNK, CHUNK)
        probs_ref[:, sl] = probs_t
        idx_ref[:, sl] = jnp.concatenate([i1, i2], axis=0)
        topk_ref[:, sl] = jnp.concatenate([p1 * rn, p2 * rn], axis=0)

        @pl.when(c + NBUF < NSTEP)
        def _refill():
            copy(c + NBUF, slot)

        return carry

    lax.fori_loop(0, NSTEP, step, 0)


@jax.jit
def kernel(x, W_gate, W_noisy):
    out_shapes = (
        jax.ShapeDtypeStruct((K, N_TOKENS), jnp.int32),
        jax.ShapeDtypeStruct((K, N_TOKENS), jnp.float32),
        jax.ShapeDtypeStruct((E, N_TOKENS), jnp.float32),
    )
    topk_idx, topk_probs, probs = pl.pallas_call(
        _body,
        in_specs=[
            pl.BlockSpec(memory_space=pl.ANY),
            pl.BlockSpec(memory_space=pltpu.VMEM),
        ],
        out_specs=(
            pl.BlockSpec(memory_space=pltpu.VMEM),
            pl.BlockSpec(memory_space=pltpu.VMEM),
            pl.BlockSpec(memory_space=pltpu.VMEM),
        ),
        out_shape=out_shapes,
        scratch_shapes=[
            pltpu.VMEM((NBUF, CHUNK, D), jnp.float32),
            pltpu.SemaphoreType.DMA((NBUF,)),
        ],
    )(x, W_gate)
    return topk_idx.T, topk_probs.T, probs.T
